# baseline (device time: 67964 ns/iter reference)
import functools

import jax
import jax.numpy as jnp
from jax import lax
from jax.experimental import pallas as pl
from jax.experimental.pallas import tpu as pltpu

N_DEV = 8
B, S, D = 2, 512, 768
HQ_LOC = 4
DH = 96
NG = 4
GR = (B * S) // NG
CHUNK = GR // 2
NPART = 3
PC = D // NPART
SCALE = 0.10206207261596577
EPS = 1e-5
MASKS = (1, 3, 4)


def _ln(h):
    m = jnp.mean(h, axis=-1, keepdims=True)
    v = jnp.mean((h - m) * (h - m), axis=-1, keepdims=True)
    return (h - m) * lax.rsqrt(v + EPS)


def _mm(a, b):
    return jnp.dot(a.astype(jnp.bfloat16), b.astype(jnp.bfloat16),
                   preferred_element_type=jnp.float32)


def _body(
    x_ref, wq_ref, wk_ref, wv_ref, wo_ref, temb_ref, wmod_ref, wff1_ref,
    wff2_ref, out_ref,
    acc, rsbuf, send_sems, recv_sems,
):
    me = lax.axis_index("i")
    b0_ = me & 1
    b1_ = (me >> 1) & 1
    b2_ = (me >> 2) & 1
    coords = (b0_ ^ b1_, b1_, b2_)

    part_axes = [tuple((p + i) % 3 for i in range(3)) for p in range(NPART)]

    barrier = pltpu.get_barrier_semaphore()
    for mask in MASKS:
        pl.semaphore_signal(
            barrier, inc=1, device_id=(me ^ mask,),
            device_id_type=pl.DeviceIdType.MESH,
        )
    pl.semaphore_wait(barrier, len(MASKS))

    def xchg(src, dst, partner, k):
        rdma = pltpu.make_async_remote_copy(
            src_ref=src, dst_ref=dst,
            send_sem=send_sems.at[k], recv_sem=recv_sems.at[k],
            device_id=(partner,), device_id_type=pl.DeviceIdType.MESH,
        )
        rdma.start()
        return rdma

    def start_step(g, s):
        rds = []
        for p in range(NPART):
            a = part_axes[p]
            c0 = coords[a[0]]
            k = (g * NPART + p) * 4 + s
            if s == 0:
                partner = me ^ MASKS[a[0]]
                src = acc.at[g, p, pl.ds(1 - c0, 1)]
                dst = rsbuf.at[g, p, pl.ds(0, 1)]
            elif s < 3:
                partner = me ^ MASKS[a[s]]
                src = acc.at[g, p, pl.ds(c0, 1)]
                dst = rsbuf.at[g, p, pl.ds(s, 1)]
            else:
                partner = me ^ MASKS[a[0]]
                src = acc.at[g, p, pl.ds(c0, 1)]
                dst = acc.at[g, p, pl.ds(c0, 1)]
            rds.append(xchg(src, dst, partner, k))
        return rds

    def finish_step(g, s, rds):
        for p, r in enumerate(rds):
            r.wait()
            if s < 3:
                c0 = coords[part_axes[p][0]]
                acc[g, p, pl.ds(c0, 1)] = (
                    acc[g, p, pl.ds(c0, 1)] + rsbuf[g, p, pl.ds(s, 1)]
                )

    active = []

    def start_ar(g):
        active.append({"g": g, "s": 0, "rds": start_step(g, 0)})

    def drive():
        for st in list(active):
            finish_step(st["g"], st["s"], st["rds"])
            st["s"] += 1
            if st["s"] < 4:
                st["rds"] = start_step(st["g"], st["s"])
            else:
                active.remove(st)

    def drain(g):
        for st in list(active):
            if st["g"] == g:
                while st["s"] < 4:
                    finish_step(st["g"], st["s"], st["rds"])
                    st["s"] += 1
                    if st["s"] < 4:
                        st["rds"] = start_step(st["g"], st["s"])
                active.remove(st)

    def store_partial(g, part):
        part = part.astype(jnp.bfloat16)
        for p in range(NPART):
            acc[g, p] = part[:, p * PC:(p + 1) * PC].reshape(2, CHUNK, PC)

    def read_full(g):
        return jnp.concatenate(
            [acc[g, p].reshape(GR, PC) for p in range(NPART)],
            axis=-1).astype(jnp.float32)

    x0 = x_ref[...]
    mod = jnp.dot(temb_ref[...], wmod_ref[...],
                  preferred_element_type=jnp.float32)
    sa, sha, ga, sm_, shm, gm = (mod[:, i * D:(i + 1) * D] for i in range(6))

    wq, wk, wv, wo = wq_ref[...], wk_ref[...], wv_ref[...], wo_ref[...]
    wff1, wff2 = wff1_ref[...], wff2_ref[...]

    def att_group(g, xa_b, kb, vb):
        r0 = (g % 2) * GR
        qg = _mm(xa_b[r0:r0 + GR], wq)
        outs = []
        for h in range(HQ_LOC):
            q = qg[:, h * DH:(h + 1) * DH]
            k = kb[:, h * DH:(h + 1) * DH]
            v = vb[:, h * DH:(h + 1) * DH]
            s_mat = _mm(q, k.T) * SCALE
            p_mat = jnp.exp(s_mat)
            l = jnp.sum(p_mat, axis=-1, keepdims=True)
            outs.append(_mm(p_mat, v) / l)
        ob = jnp.concatenate(outs, axis=-1)
        store_partial(g, _mm(ob, wo))

    xa0 = _ln(x0[0]) * (1.0 + sa[0]) + sha[0]
    k0 = _mm(xa0, wk)
    v0 = _mm(xa0, wv)
    for g in (0, 1):
        att_group(g, xa0, k0, v0)
        start_ar(g)
        drive()
    xa1 = _ln(x0[1]) * (1.0 + sa[1]) + sha[1]
    drive()
    k1 = _mm(xa1, wk)
    v1 = _mm(xa1, wv)
    drive()
    for g in (2, 3):
        att_group(g, xa1, k1, v1)
        start_ar(g)
        drive()

    x1s = [None] * NG
    for g in range(NG):
        drain(g)
        b = g // 2
        r0 = (g % 2) * GR
        attn_g = read_full(g)
        x1_g = x0[b, r0:r0 + GR] + ga[b][None, :] * attn_g
        xm_g = _ln(x1_g) * (1.0 + sm_[b][None, :]) + shm[b][None, :]
        h_g = _mm(xm_g, wff1)
        h_g = h_g / (1.0 + jnp.exp(-h_g))
        store_partial(g, _mm(h_g, wff2))
        x1s[g] = x1_g
        start_ar(g)
        drive()

    for g in range(NG):
        drain(g)
        b = g // 2
        r0 = (g % 2) * GR
        ff_g = read_full(g)
        out_ref[b, pl.ds(r0, GR), :] = x1s[g] + gm[b][None, :] * ff_g

    @functools.partial(pl.run_scoped, exit_sem=pltpu.SemaphoreType.REGULAR)
    def _(exit_sem):
        for mask in MASKS:
            pl.semaphore_signal(
                exit_sem, inc=1, device_id=(me ^ mask,),
                device_id_type=pl.DeviceIdType.MESH,
            )
        pl.semaphore_wait(exit_sem, len(MASKS))


def kernel(x, Wq, Wk, Wv, Wo, t_emb, W_mod, W_ff1, W_ff2):
    return pl.pallas_call(
        _body,
        out_shape=jax.ShapeDtypeStruct((B, S, D), jnp.float32),
        in_specs=[pl.BlockSpec(memory_space=pltpu.VMEM)] * 9,
        out_specs=pl.BlockSpec(memory_space=pltpu.VMEM),
        scratch_shapes=[
            pltpu.VMEM((NG, NPART, 2, CHUNK, PC), jnp.bfloat16),
            pltpu.VMEM((NG, NPART, 3, CHUNK, PC), jnp.bfloat16),
            pltpu.SemaphoreType.DMA((NG * NPART * 4,)),
            pltpu.SemaphoreType.DMA((NG * NPART * 4,)),
        ],
        compiler_params=pltpu.CompilerParams(collective_id=0),
    )(x, Wq, Wk, Wv, Wo, t_emb, W_mod, W_ff1, W_ff2)
